# Initial kernel scaffold; baseline (speedup 1.0000x reference)
import jax
import jax.numpy as jnp
from jax.experimental import pallas as pl


def kernel(proposals, objectness):
    raise NotImplementedError("write your pallas kernel here")


# speedup vs baseline: 2.4591x; 2.4591x over previous
import jax
import jax.numpy as jnp
from jax import lax
from jax.experimental import pallas as pl
from jax.experimental.pallas import tpu as pltpu

_B, _N = 4, 20000
_PRE_NMS = 1000
_POST_NMS = 1000
_NMS_THRESH = 0.7
_SCORE_THRESH = 0.0
_MIN_SIZE = 0.001
_IMG_H = 800.0
_IMG_W = 800.0
_K = 1024


def _nms_kernel(boxes_ref, boxest_ref, valid_ref, keep_ref, iou_ref):
    boxes = boxes_ref[0]
    bt = boxest_ref[0]
    x1 = boxes[:, 0:1]
    y1 = boxes[:, 1:2]
    x2 = boxes[:, 2:3]
    y2 = boxes[:, 3:4]
    x1c = bt[0:1, :]
    y1c = bt[1:2, :]
    x2c = bt[2:3, :]
    y2c = bt[3:4, :]
    area_r = (x2 - x1) * (y2 - y1)
    area_c = (x2c - x1c) * (y2c - y1c)
    ix1 = jnp.maximum(x1, x1c)
    iy1 = jnp.maximum(y1, y1c)
    ix2 = jnp.minimum(x2, x2c)
    iy2 = jnp.minimum(y2, y2c)
    iw = jnp.maximum(ix2 - ix1, 0.0)
    ih = jnp.maximum(iy2 - iy1, 0.0)
    inter = iw * ih
    union = area_r + area_c - inter
    iou_ref[...] = inter / jnp.maximum(union, 1e-9)

    col = lax.broadcasted_iota(jnp.int32, (1, _K), 1)
    keep_ref[0] = valid_ref[0]

    def body(i, carry):
        row = iou_ref[pl.ds(i, 1), :]
        keep = keep_ref[0]
        ki = jnp.sum(keep * (col == i).astype(jnp.float32))
        sup = (row > _NMS_THRESH) & (col > i) & (ki > 0.5)
        keep_ref[0] = keep * (1.0 - sup.astype(jnp.float32))
        return carry

    lax.fori_loop(0, _PRE_NMS, body, 0)


def kernel(proposals, objectness):
    top_scores, idx = lax.top_k(objectness, _PRE_NMS)
    boxes = jnp.take_along_axis(proposals, idx[..., None], axis=1)
    prob = jax.nn.sigmoid(top_scores)
    boxes = jnp.stack([
        jnp.clip(boxes[..., 0], 0.0, _IMG_W),
        jnp.clip(boxes[..., 1], 0.0, _IMG_H),
        jnp.clip(boxes[..., 2], 0.0, _IMG_W),
        jnp.clip(boxes[..., 3], 0.0, _IMG_H)], axis=-1)
    ws = boxes[..., 2] - boxes[..., 0]
    hs = boxes[..., 3] - boxes[..., 1]
    valid = (ws >= _MIN_SIZE) & (hs >= _MIN_SIZE) & (prob >= _SCORE_THRESH)
    prob = jnp.where(valid, prob, 0.0)
    order = jnp.argsort(-prob, axis=1)
    boxes_s = jnp.take_along_axis(boxes, order[..., None], axis=1)
    prob_s = jnp.take_along_axis(prob, order, axis=1)
    valid_s = jnp.take_along_axis(valid, order, axis=1)

    pad = _K - _PRE_NMS
    boxes_p = jnp.pad(boxes_s, ((0, 0), (0, pad), (0, 0)))
    valid_p = jnp.pad(valid_s.astype(jnp.float32), ((0, 0), (0, pad)))[:, None, :]
    boxes_t = boxes_p.transpose(0, 2, 1)

    keep_p = pl.pallas_call(
        _nms_kernel,
        grid=(_B,),
        in_specs=[
            pl.BlockSpec((1, _K, 4), lambda b: (b, 0, 0)),
            pl.BlockSpec((1, 4, _K), lambda b: (b, 0, 0)),
            pl.BlockSpec((1, 1, _K), lambda b: (b, 0, 0)),
        ],
        out_specs=pl.BlockSpec((1, 1, _K), lambda b: (b, 0, 0)),
        out_shape=jax.ShapeDtypeStruct((_B, 1, _K), jnp.float32),
        scratch_shapes=[pltpu.VMEM((_K, _K), jnp.float32)],
    )(boxes_p, boxes_t, valid_p)

    keep = keep_p[:, 0, :_PRE_NMS] > 0.5
    rank_key = jnp.where(keep, prob_s, -1.0)
    ord2 = jnp.argsort(-rank_key, axis=1)
    kept2 = jnp.take_along_axis(keep, ord2, axis=1)[:, :_POST_NMS]
    out_boxes = (jnp.take_along_axis(boxes_s, ord2[..., None], axis=1)[:, :_POST_NMS]
                 * kept2[..., None].astype(boxes.dtype))
    out_scores = (jnp.take_along_axis(prob_s, ord2, axis=1)[:, :_POST_NMS]
                  * kept2.astype(prob.dtype))
    return out_boxes, out_scores
